# baseline (speedup 1.0000x reference)
import functools

import jax
import jax.numpy as jnp
from jax.experimental import pallas as pl
from jax.experimental.pallas import tpu as pltpu

N_GRAPH = 16


def _colnorm(v):
    n2 = jnp.sum(v * v, axis=0, keepdims=True)
    return v * jax.lax.rsqrt(jnp.maximum(n2, 1e-24))


def _colcross(a, b):
    ax, ay, az = a[0:1], a[1:2], a[2:3]
    bx, by, bz = b[0:1], b[1:2], b[2:3]
    return jnp.concatenate(
        [ay * bz - az * by, az * bx - ax * bz, ax * by - ay * bx], axis=0)


def _body(n_tok, blk, n_blk, vocab,
          xT, bT, caT, cnT, ccT, scT, emb, W1a, W1tail, W2, b2,
          Wout, bout, Wout1, bout1,
          out, out1, moe,
          geo, w1c, ohg, cnts, pooled, moeacc, woutv, wout1v, sem0, sem1):
    i = pl.program_id(0)
    G = N_GRAPH

    @pl.when(i == 0)
    def _init():
        pltpu.make_async_copy(Wout, woutv, sem0).start()
        pltpu.make_async_copy(Wout1, wout1v, sem1).start()

        w1e = jnp.dot(emb[:], W1a[:], preferred_element_type=jnp.float32,
                      precision=jax.lax.Precision.HIGHEST)
        w1c[:] = jnp.concatenate(
            [w1e.astype(jnp.bfloat16), W1tail[:]], axis=0)

        bt = bT[:]
        g_iota = jax.lax.broadcasted_iota(jnp.int32, (G, n_tok), 0)
        ohgf = (g_iota == bt).astype(jnp.float32)
        ohg[:] = ohgf.astype(jnp.bfloat16)
        c = jnp.sum(ohgf, axis=1, keepdims=True)
        cnts[:] = c
        row = jax.lax.broadcasted_iota(jnp.int32, (G, G), 0)
        col = jax.lax.broadcasted_iota(jnp.int32, (G, G), 1)
        tri = (col < row).astype(jnp.float32)
        offs = jnp.dot(tri, c, preferred_element_type=jnp.float32,
                       precision=jax.lax.Precision.HIGHEST)
        offs_tok = jnp.sum(offs * ohgf, axis=0, keepdims=True)
        seq = (jax.lax.broadcasted_iota(jnp.int32, (1, n_tok), 1)
               .astype(jnp.float32) - offs_tok + 1.0)
        seq_hi = jnp.floor(seq * (1.0 / 256.0)) * 256.0
        seq_lo = seq - seq_hi

        v_iota = jax.lax.broadcasted_iota(jnp.int32, (vocab, n_tok), 0)
        ohx = (v_iota == xT[:]).astype(jnp.float32)

        pos = caT[:]
        u = _colnorm(pos[:, 1:] - pos[:, :-1])
        u1 = u[:, 1:]
        u2 = u[:, :-1]
        bvec = _colnorm(u2 - u1)
        nvec = _colnorm(_colcross(u2, u1))
        ovec = _colnorm(_colcross(bvec, nvec))
        interior = jnp.concatenate([bvec, nvec, ovec], axis=0)
        ori9 = jnp.concatenate(
            [interior[:, :1], interior, interior[:, -1:]], axis=1)

        ca = pos
        dn = cnT[:] - ca
        dc = ccT[:] - ca
        geo[:] = jnp.concatenate(
            [ohx, ori9, scT[:], ca, dn, dc, seq_hi, seq_lo,
             jnp.ones((1, n_tok), jnp.float32),
             jnp.zeros((64 - vocab - 29, n_tok), jnp.float32)],
            axis=0).astype(jnp.bfloat16)

        pooled[:] = jnp.zeros_like(pooled)
        moeacc[:] = jnp.zeros_like(moeacc)

    s = i * blk
    gb = geo[:, pl.ds(s, blk)]
    h1 = jax.lax.dot_general(gb, w1c[:], (((0,), (0,)), ((), ())),
                             preferred_element_type=jnp.float32)
    h1 = jnp.maximum(h1, 0.0)
    h2 = jnp.maximum(
        jnp.dot(h1.astype(jnp.bfloat16), W2[:],
                preferred_element_type=jnp.float32) + b2[:], 0.0)
    h2b = h2.astype(jnp.bfloat16)

    og = ohg[:, pl.ds(s, blk)]
    pooled[:] += jax.lax.dot_general(
        og, h2b, (((1,), (0,)), ((), ())),
        preferred_element_type=jnp.float32)
    sq = h2b * h2b
    ones_row = jnp.ones((1, blk), jnp.bfloat16)
    moeacc[:] += jax.lax.dot_general(
        ones_row, sq, (((1,), (0,)), ((), ())),
        preferred_element_type=jnp.float32)

    @pl.when(i == n_blk - 1)
    def _fin():
        pm = (pooled[:] / jnp.maximum(cnts[:], 1.0)).astype(jnp.bfloat16)
        pltpu.make_async_copy(Wout, woutv, sem0).wait()
        pltpu.make_async_copy(Wout1, wout1v, sem1).wait()
        out[:] = jnp.dot(pm, woutv[:],
                         preferred_element_type=jnp.float32) + bout[:]
        out1[:] = jnp.dot(pm, wout1v[:],
                          preferred_element_type=jnp.float32) + bout1[:]
        moe[:] = jnp.sum(moeacc[:]).reshape(1, 1) * (0.01 / (n_tok * W2.shape[1]))


def kernel(x, coords_ca, coords_n, coords_c, batch, side_chain_embs, istrain,
           emb, W1, b1, W2, b2, Wout, bout, Wout1, bout1):
    n_tok = x.shape[0]
    vocab, embdim = emb.shape
    h1d = W1.shape[1]
    h2d = W2.shape[1]
    ncls = Wout.shape[1]
    blk = 2048
    n_blk = n_tok // blk

    xT = x.reshape(1, n_tok).astype(jnp.int32)
    bT = batch.reshape(1, n_tok).astype(jnp.int32)
    caT = coords_ca.T
    cnT = coords_n.T
    ccT = coords_c.T
    scT = side_chain_embs.T
    W1a = W1[:embdim]
    W1tail = jnp.concatenate(
        [W1[embdim:], W1[-1:], b1.reshape(1, h1d),
         jnp.zeros((64 - vocab - 29, h1d), W1.dtype)],
        axis=0).astype(jnp.bfloat16)

    full = lambda a: pl.BlockSpec(a.shape, lambda i: (0,) * a.ndim)
    hbm = pl.BlockSpec(memory_space=pltpu.MemorySpace.HBM)
    inputs = (xT, bT, caT, cnT, ccT, scT, emb, W1a, W1tail,
              W2.astype(jnp.bfloat16), b2.reshape(1, h2d),
              Wout.astype(jnp.bfloat16), bout.reshape(1, ncls),
              Wout1.astype(jnp.bfloat16), bout1.reshape(1, ncls))
    in_specs = [full(a) for a in inputs]
    in_specs[11] = hbm
    in_specs[13] = hbm

    out, out1, moe = pl.pallas_call(
        functools.partial(_body, n_tok, blk, n_blk, vocab),
        grid=(n_blk,),
        in_specs=in_specs,
        out_specs=[
            pl.BlockSpec((N_GRAPH, ncls), lambda i: (0, 0)),
            pl.BlockSpec((N_GRAPH, ncls), lambda i: (0, 0)),
            pl.BlockSpec((1, 1), lambda i: (0, 0)),
        ],
        out_shape=[
            jax.ShapeDtypeStruct((N_GRAPH, ncls), jnp.float32),
            jax.ShapeDtypeStruct((N_GRAPH, ncls), jnp.float32),
            jax.ShapeDtypeStruct((1, 1), jnp.float32),
        ],
        scratch_shapes=[
            pltpu.VMEM((64, n_tok), jnp.bfloat16),
            pltpu.VMEM((64, h1d), jnp.bfloat16),
            pltpu.VMEM((N_GRAPH, n_tok), jnp.bfloat16),
            pltpu.VMEM((N_GRAPH, 1), jnp.float32),
            pltpu.VMEM((N_GRAPH, h2d), jnp.float32),
            pltpu.VMEM((1, h2d), jnp.float32),
            pltpu.VMEM((h2d, ncls), jnp.bfloat16),
            pltpu.VMEM((h2d, ncls), jnp.bfloat16),
            pltpu.SemaphoreType.DMA,
            pltpu.SemaphoreType.DMA,
        ],
        compiler_params=pltpu.CompilerParams(
            dimension_semantics=("arbitrary",)),
    )(*inputs)
    return (out, out1, moe.reshape(()))


# speedup vs baseline: 1.0867x; 1.0867x over previous
import functools

import jax
import jax.numpy as jnp
from jax.experimental import pallas as pl
from jax.experimental.pallas import tpu as pltpu

N_GRAPH = 16


def _colnorm(v):
    n2 = jnp.sum(v * v, axis=0, keepdims=True)
    return v * jax.lax.rsqrt(jnp.maximum(n2, 1e-24))


def _colcross(a, b):
    ax, ay, az = a[0:1], a[1:2], a[2:3]
    bx, by, bz = b[0:1], b[1:2], b[2:3]
    return jnp.concatenate(
        [ay * bz - az * by, az * bx - ax * bz, ax * by - ay * bx], axis=0)


def _body(n_tok, blk, n_blk, vocab,
          xT, bT, caT, cnT, ccT, scT, emb, W1a, W1tail, W2, b2,
          Wout, bout, Wout1, bout1,
          out, out1, moe,
          geo, w1c, ohg, cnts, pooled, moeacc, woutv, wout1v, sem0, sem1):
    i = pl.program_id(0)
    G = N_GRAPH

    @pl.when(i == 0)
    def _init():
        pltpu.make_async_copy(Wout, woutv, sem0).start()
        pltpu.make_async_copy(Wout1, wout1v, sem1).start()

        w1e = jnp.dot(emb[:], W1a[:], preferred_element_type=jnp.float32,
                      precision=jax.lax.Precision.HIGHEST)
        w1c[:] = jnp.concatenate(
            [w1e.astype(jnp.bfloat16), W1tail[:]], axis=0)

        bt = bT[:]
        g_iota = jax.lax.broadcasted_iota(jnp.int32, (G, n_tok), 0)
        ohgf = (g_iota == bt).astype(jnp.float32)
        ohg[:] = ohgf.astype(jnp.bfloat16)
        c = jnp.sum(ohgf, axis=1, keepdims=True)
        cnts[:] = c
        row = jax.lax.broadcasted_iota(jnp.int32, (G, G), 0)
        col = jax.lax.broadcasted_iota(jnp.int32, (G, G), 1)
        tri = (col < row).astype(jnp.float32)
        offs = jnp.dot(tri, c, preferred_element_type=jnp.float32,
                       precision=jax.lax.Precision.HIGHEST)
        offs_tok = jnp.sum(offs * ohgf, axis=0, keepdims=True)
        seq = (jax.lax.broadcasted_iota(jnp.int32, (1, n_tok), 1)
               .astype(jnp.float32) - offs_tok + 1.0)
        seq_hi = jnp.floor(seq * (1.0 / 256.0)) * 256.0
        seq_lo = seq - seq_hi

        v_iota = jax.lax.broadcasted_iota(jnp.int32, (vocab, n_tok), 0)
        ohx = (v_iota == xT[:]).astype(jnp.float32)

        pos = caT[:]
        u = _colnorm(pos[:, 1:] - pos[:, :-1])
        u1 = u[:, 1:]
        u2 = u[:, :-1]
        bvec = _colnorm(u2 - u1)
        nvec = _colnorm(_colcross(u2, u1))
        ovec = _colnorm(_colcross(bvec, nvec))
        interior = jnp.concatenate([bvec, nvec, ovec], axis=0)
        ori9 = jnp.concatenate(
            [interior[:, :1], interior, interior[:, -1:]], axis=1)

        ca = pos
        dn = cnT[:] - ca
        dc = ccT[:] - ca
        geo[:] = jnp.concatenate(
            [ohx, ori9, scT[:], ca, dn, dc, seq_hi, seq_lo,
             jnp.ones((1, n_tok), jnp.float32),
             jnp.zeros((64 - vocab - 29, n_tok), jnp.float32)],
            axis=0).astype(jnp.bfloat16)

        pooled[:] = jnp.zeros_like(pooled)
        moeacc[:] = jnp.zeros_like(moeacc)

    s = i * blk
    gb = geo[:, pl.ds(s, blk)]
    h1 = jax.lax.dot_general(gb, w1c[:], (((0,), (0,)), ((), ())),
                             preferred_element_type=jnp.float32)
    h1 = jnp.maximum(h1, 0.0)
    h2 = jnp.maximum(
        jnp.dot(h1.astype(jnp.bfloat16), W2[:],
                preferred_element_type=jnp.float32) + b2[:], 0.0)
    h2b = h2.astype(jnp.bfloat16)

    og = ohg[:, pl.ds(s, blk)]
    pooled[:] += jax.lax.dot_general(
        og, h2b, (((1,), (0,)), ((), ())),
        preferred_element_type=jnp.float32)
    moeacc[:] += jnp.sum(h2 * h2, axis=0, keepdims=True)

    @pl.when(i == n_blk - 1)
    def _fin():
        pm = (pooled[:] / jnp.maximum(cnts[:], 1.0)).astype(jnp.bfloat16)
        pltpu.make_async_copy(Wout, woutv, sem0).wait()
        pltpu.make_async_copy(Wout1, wout1v, sem1).wait()
        out[:] = jnp.dot(pm, woutv[:],
                         preferred_element_type=jnp.float32) + bout[:]
        out1[:] = jnp.dot(pm, wout1v[:],
                          preferred_element_type=jnp.float32) + bout1[:]
        moe[:] = jnp.sum(moeacc[:]).reshape(1, 1) * (0.01 / (n_tok * W2.shape[1]))


def kernel(x, coords_ca, coords_n, coords_c, batch, side_chain_embs, istrain,
           emb, W1, b1, W2, b2, Wout, bout, Wout1, bout1):
    n_tok = x.shape[0]
    vocab, embdim = emb.shape
    h1d = W1.shape[1]
    h2d = W2.shape[1]
    ncls = Wout.shape[1]
    blk = 2048
    n_blk = n_tok // blk

    xT = x.reshape(1, n_tok).astype(jnp.int32)
    bT = batch.reshape(1, n_tok).astype(jnp.int32)
    caT = coords_ca.T
    cnT = coords_n.T
    ccT = coords_c.T
    scT = side_chain_embs.T
    W1a = W1[:embdim]
    W1tail = jnp.concatenate(
        [W1[embdim:], W1[-1:], b1.reshape(1, h1d),
         jnp.zeros((64 - vocab - 29, h1d), W1.dtype)],
        axis=0).astype(jnp.bfloat16)

    full = lambda a: pl.BlockSpec(a.shape, lambda i: (0,) * a.ndim)
    hbm = pl.BlockSpec(memory_space=pltpu.MemorySpace.HBM)
    inputs = (xT, bT, caT, cnT, ccT, scT, emb, W1a, W1tail,
              W2.astype(jnp.bfloat16), b2.reshape(1, h2d),
              Wout.astype(jnp.bfloat16), bout.reshape(1, ncls),
              Wout1.astype(jnp.bfloat16), bout1.reshape(1, ncls))
    in_specs = [full(a) for a in inputs]
    in_specs[11] = hbm
    in_specs[13] = hbm

    out, out1, moe = pl.pallas_call(
        functools.partial(_body, n_tok, blk, n_blk, vocab),
        grid=(n_blk,),
        in_specs=in_specs,
        out_specs=[
            pl.BlockSpec((N_GRAPH, ncls), lambda i: (0, 0)),
            pl.BlockSpec((N_GRAPH, ncls), lambda i: (0, 0)),
            pl.BlockSpec((1, 1), lambda i: (0, 0)),
        ],
        out_shape=[
            jax.ShapeDtypeStruct((N_GRAPH, ncls), jnp.float32),
            jax.ShapeDtypeStruct((N_GRAPH, ncls), jnp.float32),
            jax.ShapeDtypeStruct((1, 1), jnp.float32),
        ],
        scratch_shapes=[
            pltpu.VMEM((64, n_tok), jnp.bfloat16),
            pltpu.VMEM((64, h1d), jnp.bfloat16),
            pltpu.VMEM((N_GRAPH, n_tok), jnp.bfloat16),
            pltpu.VMEM((N_GRAPH, 1), jnp.float32),
            pltpu.VMEM((N_GRAPH, h2d), jnp.float32),
            pltpu.VMEM((1, h2d), jnp.float32),
            pltpu.VMEM((h2d, ncls), jnp.bfloat16),
            pltpu.VMEM((h2d, ncls), jnp.bfloat16),
            pltpu.SemaphoreType.DMA,
            pltpu.SemaphoreType.DMA,
        ],
        compiler_params=pltpu.CompilerParams(
            dimension_semantics=("arbitrary",)),
    )(*inputs)
    return (out, out1, moe.reshape(()))


# speedup vs baseline: 1.0931x; 1.0059x over previous
import functools

import jax
import jax.numpy as jnp
from jax.experimental import pallas as pl
from jax.experimental.pallas import tpu as pltpu

N_GRAPH = 16


def _colnorm(v):
    n = jnp.sqrt(jnp.sum(v * v, axis=0, keepdims=True))
    return v / jnp.maximum(n, 1e-12)


def _colcross(a, b):
    ax, ay, az = a[0:1], a[1:2], a[2:3]
    bx, by, bz = b[0:1], b[1:2], b[2:3]
    return jnp.concatenate(
        [ay * bz - az * by, az * bx - ax * bz, ax * by - ay * bx], axis=0)


def _body(n_tok, blk, n_blk, vocab,
          xT, bT, caT, cnT, ccT, scT, emb, W1a, W1tail, b1, W2, b2,
          Wout, bout, Wout1, bout1,
          out, out1, moe,
          geo, w1c, ohg, cnts, pooled, moeacc, woutv, wout1v, sem0, sem1):
    i = pl.program_id(0)
    G = N_GRAPH

    @pl.when(i == 0)
    def _init():
        pltpu.make_async_copy(Wout, woutv, sem0).start()
        pltpu.make_async_copy(Wout1, wout1v, sem1).start()

        w1e = jnp.dot(emb[:], W1a[:], preferred_element_type=jnp.float32,
                      precision=jax.lax.Precision.HIGHEST)
        w1c[:] = jnp.concatenate(
            [w1e.astype(jnp.bfloat16), W1tail[:]], axis=0)

        bt = bT[:]
        g_iota = jax.lax.broadcasted_iota(jnp.int32, (G, n_tok), 0)
        ohgf = (g_iota == bt).astype(jnp.float32)
        ohg[:] = ohgf.astype(jnp.bfloat16)
        c = jnp.sum(ohgf, axis=1, keepdims=True)
        cnts[:] = c
        row = jax.lax.broadcasted_iota(jnp.int32, (G, G), 0)
        col = jax.lax.broadcasted_iota(jnp.int32, (G, G), 1)
        tri = (col < row).astype(jnp.float32)
        offs = jnp.dot(tri, c, preferred_element_type=jnp.float32,
                       precision=jax.lax.Precision.HIGHEST)
        offs_tok = jnp.sum(offs * ohgf, axis=0, keepdims=True)
        seq = (jax.lax.broadcasted_iota(jnp.int32, (1, n_tok), 1)
               .astype(jnp.float32) - offs_tok + 1.0)
        seq_hi = jnp.floor(seq * (1.0 / 256.0)) * 256.0
        seq_lo = seq - seq_hi

        v_iota = jax.lax.broadcasted_iota(jnp.int32, (vocab, n_tok), 0)
        ohx = (v_iota == xT[:]).astype(jnp.float32)

        pos = caT[:]
        u = _colnorm(pos[:, 1:] - pos[:, :-1])
        u1 = u[:, 1:]
        u2 = u[:, :-1]
        bvec = _colnorm(u2 - u1)
        nvec = _colnorm(_colcross(u2, u1))
        ovec = _colnorm(_colcross(bvec, nvec))
        interior = jnp.concatenate([bvec, nvec, ovec], axis=0)
        ori9 = jnp.concatenate(
            [interior[:, :1], interior, interior[:, -1:]], axis=1)

        ca = pos
        dn = cnT[:] - ca
        dc = ccT[:] - ca
        geo[:] = jnp.concatenate(
            [ohx, ori9, scT[:], ca, dn, dc, seq_hi, seq_lo,
             jnp.zeros((64 - vocab - 28, n_tok), jnp.float32)],
            axis=0).astype(jnp.bfloat16)

        pooled[:] = jnp.zeros_like(pooled)
        moeacc[:] = jnp.zeros_like(moeacc)

    s = i * blk
    gb = geo[:, pl.ds(s, blk)]
    h1 = jax.lax.dot_general(gb, w1c[:], (((0,), (0,)), ((), ())),
                             preferred_element_type=jnp.float32)
    h1 = jnp.maximum(h1 + b1[:], 0.0)
    h2 = jnp.maximum(
        jnp.dot(h1.astype(jnp.bfloat16), W2[:],
                preferred_element_type=jnp.float32) + b2[:], 0.0)

    og = ohg[:, pl.ds(s, blk)]
    pooled[:] += jax.lax.dot_general(
        og, h2.astype(jnp.bfloat16), (((1,), (0,)), ((), ())),
        preferred_element_type=jnp.float32)
    moeacc[:] += jnp.sum(h2 * h2, axis=0, keepdims=True)

    @pl.when(i == n_blk - 1)
    def _fin():
        pm = (pooled[:] / jnp.maximum(cnts[:], 1.0)).astype(jnp.bfloat16)
        pltpu.make_async_copy(Wout, woutv, sem0).wait()
        pltpu.make_async_copy(Wout1, wout1v, sem1).wait()
        out[:] = jnp.dot(pm, woutv[:],
                         preferred_element_type=jnp.float32) + bout[:]
        out1[:] = jnp.dot(pm, wout1v[:],
                          preferred_element_type=jnp.float32) + bout1[:]
        moe[:] = jnp.sum(moeacc[:]).reshape(1, 1) * (0.01 / (n_tok * W2.shape[1]))


def kernel(x, coords_ca, coords_n, coords_c, batch, side_chain_embs, istrain,
           emb, W1, b1, W2, b2, Wout, bout, Wout1, bout1):
    n_tok = x.shape[0]
    vocab, embdim = emb.shape
    h1d = W1.shape[1]
    h2d = W2.shape[1]
    ncls = Wout.shape[1]
    blk = 2048
    n_blk = n_tok // blk

    xT = x.reshape(1, n_tok).astype(jnp.int32)
    bT = batch.reshape(1, n_tok).astype(jnp.int32)
    caT = coords_ca.T
    cnT = coords_n.T
    ccT = coords_c.T
    scT = side_chain_embs.T
    W1a = W1[:embdim]
    W1tail = jnp.concatenate(
        [W1[embdim:], W1[-1:], jnp.zeros((64 - vocab - 28, h1d), W1.dtype)],
        axis=0).astype(jnp.bfloat16)

    full = lambda a: pl.BlockSpec(a.shape, lambda i: (0,) * a.ndim)
    hbm = pl.BlockSpec(memory_space=pltpu.MemorySpace.HBM)
    inputs = (xT, bT, caT, cnT, ccT, scT, emb, W1a, W1tail,
              b1.reshape(1, h1d), W2.astype(jnp.bfloat16),
              b2.reshape(1, h2d),
              Wout.astype(jnp.bfloat16), bout.reshape(1, ncls),
              Wout1.astype(jnp.bfloat16), bout1.reshape(1, ncls))
    in_specs = [full(a) for a in inputs]
    in_specs[12] = hbm
    in_specs[14] = hbm

    out, out1, moe = pl.pallas_call(
        functools.partial(_body, n_tok, blk, n_blk, vocab),
        grid=(n_blk,),
        in_specs=in_specs,
        out_specs=[
            pl.BlockSpec((N_GRAPH, ncls), lambda i: (0, 0)),
            pl.BlockSpec((N_GRAPH, ncls), lambda i: (0, 0)),
            pl.BlockSpec((1, 1), lambda i: (0, 0)),
        ],
        out_shape=[
            jax.ShapeDtypeStruct((N_GRAPH, ncls), jnp.float32),
            jax.ShapeDtypeStruct((N_GRAPH, ncls), jnp.float32),
            jax.ShapeDtypeStruct((1, 1), jnp.float32),
        ],
        scratch_shapes=[
            pltpu.VMEM((64, n_tok), jnp.bfloat16),
            pltpu.VMEM((64, h1d), jnp.bfloat16),
            pltpu.VMEM((N_GRAPH, n_tok), jnp.bfloat16),
            pltpu.VMEM((N_GRAPH, 1), jnp.float32),
            pltpu.VMEM((N_GRAPH, h2d), jnp.float32),
            pltpu.VMEM((1, h2d), jnp.float32),
            pltpu.VMEM((h2d, ncls), jnp.bfloat16),
            pltpu.VMEM((h2d, ncls), jnp.bfloat16),
            pltpu.SemaphoreType.DMA,
            pltpu.SemaphoreType.DMA,
        ],
        compiler_params=pltpu.CompilerParams(
            dimension_semantics=("arbitrary",)),
    )(*inputs)
    return (out, out1, moe.reshape(()))
